# Initial kernel scaffold; baseline (speedup 1.0000x reference)
import jax
import jax.numpy as jnp
from jax.experimental import pallas as pl


def kernel(x, src, dst, w, Win, b_in, Wm, bm, Wu, bu, Wos, bos, Wside, bside):
    raise NotImplementedError("write your pallas kernel here")


# speedup vs baseline: 2.9359x; 2.9359x over previous
import functools

import jax
import jax.numpy as jnp
from jax import lax
from jax.experimental import pallas as pl
from jax.experimental.pallas import tpu as pltpu
from jax.experimental.pallas import tpu_sc as plsc

N = 10000
E = 320000
D = 128
H = 128
L = 3

NUM_CORES = 2
NUM_SUBCORES = 16
NUM_WORKERS = NUM_CORES * NUM_SUBCORES
GROUP = 128
G_PER_W = 80
EP = NUM_WORKERS * G_PER_W * GROUP
ROWS_PER_SUB = 624
ROW_TAIL = N - NUM_SUBCORES * ROWS_PER_SUB
ROW_TAIL_OFF = NUM_SUBCORES * ROWS_PER_SUB

_TC_BLOCK = 1000


def _sc_body(m_hbm, src_hbm, dst_hbm, w_hbm, zero_hbm,
             out_hbm, acc, src_v, dst_v, att_v, rows_v, sem):
    c = lax.axis_index("c")
    s = lax.axis_index("s")
    wid = s * NUM_CORES + c

    pltpu.sync_copy(zero_hbm.at[pl.ds(0, ROWS_PER_SUB)],
                    acc.at[pl.ds(s * ROWS_PER_SUB, ROWS_PER_SUB)])

    @pl.when(s == NUM_SUBCORES - 1)
    def _zero_tail():
        pltpu.sync_copy(zero_hbm.at[pl.ds(0, ROW_TAIL)],
                        acc.at[pl.ds(ROW_TAIL_OFF, ROW_TAIL)])

    base = wid * G_PER_W
    pltpu.sync_copy(src_hbm.at[pl.ds(base, G_PER_W)], src_v)
    pltpu.sync_copy(dst_hbm.at[pl.ds(base, G_PER_W)], dst_v)
    pltpu.sync_copy(w_hbm.at[pl.ds(base, G_PER_W)], att_v)

    def sig_body(j, carry):
        g = j // 8
        f = (j % 8) * 16
        wv = att_v[g, pl.ds(f, 16)]
        att_v[g, pl.ds(f, 16)] = 1.0 / (1.0 + jnp.exp(-wv))
        return carry

    lax.fori_loop(0, G_PER_W * 8, sig_body, 0)

    plsc.subcore_barrier()

    def group_body(g, carry):
        pltpu.async_copy(m_hbm.at[src_v.at[g]], rows_v, sem).wait()

        def edge_chunk_body(cidx, inner):
            e0 = cidx * 16
            av = att_v[g, pl.ds(e0, 16)]
            for j in range(16):
                a = av[j]
                for f in range(8):
                    sl = pl.ds(f * 16, 16)
                    rows_v[e0 + j, sl] = rows_v[e0 + j, sl] * a
            return inner

        lax.fori_loop(0, GROUP // 16, edge_chunk_body, 0)

        pltpu.sync_copy(rows_v, acc.at[dst_v.at[g]], add=True)
        return carry

    lax.fori_loop(0, G_PER_W, group_body, 0)

    plsc.subcore_barrier()
    pltpu.sync_copy(acc.at[pl.ds(s * ROWS_PER_SUB, ROWS_PER_SUB)],
                    out_hbm.at[c, pl.ds(s * ROWS_PER_SUB, ROWS_PER_SUB)])

    @pl.when(s == NUM_SUBCORES - 1)
    def _write_tail():
        pltpu.sync_copy(acc.at[pl.ds(ROW_TAIL_OFF, ROW_TAIL)],
                        out_hbm.at[c, pl.ds(ROW_TAIL_OFF, ROW_TAIL)])


_sc_scatter = functools.partial(
    pl.kernel,
    out_type=jax.ShapeDtypeStruct((NUM_CORES, N, H), jnp.float32),
    mesh=plsc.VectorSubcoreMesh(core_axis_name="c", subcore_axis_name="s"),
    scratch_types=[
        pltpu.VMEM_SHARED((N, H), jnp.float32),
        pltpu.VMEM((G_PER_W, GROUP), jnp.int32),
        pltpu.VMEM((G_PER_W, GROUP), jnp.int32),
        pltpu.VMEM((G_PER_W, GROUP), jnp.float32),
        pltpu.VMEM((GROUP, H), jnp.float32),
        pltpu.SemaphoreType.DMA,
    ],
)(_sc_body)


def _mm(a, b):
    return jax.lax.dot_general(a, b, (((1,), (0,)), ((), ())),
                               preferred_element_type=jnp.float32)


def _tc_in_body(x_ref, win_ref, bin_ref, wm_ref, bm_ref, h_ref, m_ref):
    h = jnp.maximum(_mm(x_ref[...], win_ref[...]) + bin_ref[...], 0.0)
    h_ref[...] = h
    m_ref[...] = _mm(h, wm_ref[...]) + bm_ref[...]


def _tc_mid_body(h_ref, p0_ref, p1_ref, wu_ref, bu_ref, wm_ref, bm_ref,
                 h2_ref, m2_ref):
    t = h_ref[...] + p0_ref[...] + p1_ref[...]
    h2 = jnp.maximum(_mm(t, wu_ref[...]) + bu_ref[...], 0.0)
    h2_ref[...] = h2
    m2_ref[...] = _mm(h2, wm_ref[...]) + bm_ref[...]


def _tc_out_body(h_ref, p0_ref, p1_ref, wu_ref, bu_ref, who_ref, bho_ref,
                 z_ref):
    t = h_ref[...] + p0_ref[...] + p1_ref[...]
    h3 = jnp.maximum(_mm(t, wu_ref[...]) + bu_ref[...], 0.0)
    z = _mm(h3, who_ref[...]) + bho_ref[...]
    col = lax.broadcasted_iota(jnp.int32, z.shape, 1)
    z_ref[...] = jnp.where(col == 0, jax.nn.sigmoid(z), jnp.tanh(z))


def _row_spec(block):
    return pl.BlockSpec((block, H), lambda i: (i, 0))


_FULL_W = pl.BlockSpec((H, H), lambda i: (0, 0))
_FULL_B = pl.BlockSpec((1, H), lambda i: (0, 0))
_GRID = (N // _TC_BLOCK,)

_tc_in = pl.pallas_call(
    _tc_in_body,
    grid=_GRID,
    in_specs=[_row_spec(_TC_BLOCK), _FULL_W, _FULL_B, _FULL_W, _FULL_B],
    out_specs=[_row_spec(_TC_BLOCK), _row_spec(_TC_BLOCK)],
    out_shape=[jax.ShapeDtypeStruct((N, H), jnp.float32),
               jax.ShapeDtypeStruct((N, H), jnp.float32)],
)

_tc_mid = pl.pallas_call(
    _tc_mid_body,
    grid=_GRID,
    in_specs=[_row_spec(_TC_BLOCK)] * 3 + [_FULL_W, _FULL_B, _FULL_W, _FULL_B],
    out_specs=[_row_spec(_TC_BLOCK), _row_spec(_TC_BLOCK)],
    out_shape=[jax.ShapeDtypeStruct((N, H), jnp.float32),
               jax.ShapeDtypeStruct((N, H), jnp.float32)],
)

_tc_out = pl.pallas_call(
    _tc_out_body,
    grid=_GRID,
    in_specs=[_row_spec(_TC_BLOCK)] * 3 + [_FULL_W, _FULL_B, _FULL_W, _FULL_B],
    out_specs=[_row_spec(_TC_BLOCK)],
    out_shape=[jax.ShapeDtypeStruct((N, H), jnp.float32)],
)


def kernel(x, src, dst, w, Win, b_in, Wm, bm, Wu, bu, Wos, bos, Wside, bside):
    pad = EP - E
    srcp = jnp.concatenate([src.astype(jnp.int32),
                            jnp.zeros((pad,), jnp.int32)]).reshape(-1, GROUP)
    dstp = jnp.concatenate([dst.astype(jnp.int32),
                            jnp.zeros((pad,), jnp.int32)]).reshape(-1, GROUP)
    wp = jnp.concatenate([w, jnp.full((pad,), -1e30, jnp.float32)]
                         ).reshape(-1, GROUP)
    zero_rows = jnp.zeros((ROWS_PER_SUB, H), jnp.float32)

    bin2 = b_in.reshape(1, H)
    who = jnp.zeros((H, H), jnp.float32)
    who = who.at[:, 0].set(Wos[:, 0]).at[:, 1].set(Wside[:, 0])
    bho = jnp.zeros((H,), jnp.float32).at[0].set(bos[0]).at[1].set(bside[0])
    bho = bho.reshape(1, H)

    h, m = _tc_in(x, Win, bin2, Wm[0], bm[0].reshape(1, H))

    for k in range(L):
        partials = _sc_scatter(m, srcp, dstp, wp, zero_rows)
        p0, p1 = partials[0], partials[1]
        if k < L - 1:
            h, m = _tc_mid(h, p0, p1, Wu[k], bu[k].reshape(1, H),
                           Wm[k + 1], bm[k + 1].reshape(1, H))
        else:
            z, = _tc_out(h, p0, p1, Wu[k], bu[k].reshape(1, H), who, bho)

    return (z[:, 0], z[:, 1])


# speedup vs baseline: 3.2709x; 1.1141x over previous
import functools

import jax
import jax.numpy as jnp
from jax import lax
from jax.experimental import pallas as pl
from jax.experimental.pallas import tpu as pltpu
from jax.experimental.pallas import tpu_sc as plsc

N = 10000
E = 320000
D = 128
H = 128
L = 3

NUM_CORES = 2
NUM_SUBCORES = 16
NUM_WORKERS = NUM_CORES * NUM_SUBCORES
GROUP = 64
G_STAGE = 32
N_STAGE = 5
G_PER_W = G_STAGE * N_STAGE
NBUF = 3
TRI = (G_STAGE - 2) // NBUF
EP = NUM_WORKERS * G_PER_W * GROUP
ROWS_PER_SUB = 624
ROW_TAIL = N - NUM_SUBCORES * ROWS_PER_SUB
ROW_TAIL_OFF = NUM_SUBCORES * ROWS_PER_SUB

_TC_BLOCK = 1000


def _sc_body(m_hbm, src_hbm, dst_hbm, w_hbm, zero_hbm,
             out_hbm, acc, src_v, dst_v, att_v, rows_v,
             gs0, gs1, gs2, ss0, ss1, ss2):
    gsems = (gs0, gs1, gs2)
    ssems = (ss0, ss1, ss2)
    c = lax.axis_index("c")
    s = lax.axis_index("s")
    wid = s * NUM_CORES + c

    pltpu.sync_copy(zero_hbm.at[pl.ds(0, ROWS_PER_SUB)],
                    acc.at[pl.ds(s * ROWS_PER_SUB, ROWS_PER_SUB)])

    @pl.when(s == NUM_SUBCORES - 1)
    def _zero_tail():
        pltpu.sync_copy(zero_hbm.at[pl.ds(0, ROW_TAIL)],
                        acc.at[pl.ds(ROW_TAIL_OFF, ROW_TAIL)])

    def start_gather(g, b):
        pltpu.async_copy(m_hbm.at[src_v.at[g]], rows_v.at[b], gsems[b])

    def wait_gather(b):
        pltpu.make_async_copy(m_hbm.at[src_v.at[0]], rows_v.at[b],
                              gsems[b]).wait()

    def start_scatter(g, b):
        pltpu.async_copy(rows_v.at[b], acc.at[dst_v.at[g]], ssems[b],
                         add=True)

    def wait_scatter(b):
        pltpu.make_async_copy(rows_v.at[b], acc.at[dst_v.at[0]],
                              ssems[b]).wait()

    def scale(g, b):
        def edge_chunk_body(cidx, inner):
            e0 = cidx * 16
            av = att_v[g, pl.ds(e0, 16)]
            for j in range(16):
                a = av[j]
                for f in range(H // 16):
                    sl = pl.ds(f * 16, 16)
                    rows_v[b, e0 + j, sl] = rows_v[b, e0 + j, sl] * a
            return inner

        lax.fori_loop(0, GROUP // 16, edge_chunk_body, 0)

    def run_stage(hs):
        base = wid * G_PER_W + hs * G_STAGE
        pltpu.sync_copy(src_hbm.at[pl.ds(base, G_STAGE)], src_v)
        pltpu.sync_copy(dst_hbm.at[pl.ds(base, G_STAGE)], dst_v)
        pltpu.sync_copy(w_hbm.at[pl.ds(base, G_STAGE)], att_v)

        n_sl = GROUP // 16

        def sig_body(j, carry):
            g = j // n_sl
            f = (j % n_sl) * 16
            wv = att_v[g, pl.ds(f, 16)]
            att_v[g, pl.ds(f, 16)] = 1.0 / (1.0 + jnp.exp(-wv))
            return carry

        lax.fori_loop(0, G_STAGE * n_sl, sig_body, 0)

        start_gather(0, 0)
        start_gather(1, 1)

        def tri_body(i, carry):
            for b in range(NBUF):
                g = 3 * i + b
                nb = (b + 2) % 3
                if b == 0:
                    @pl.when(i > 0)
                    def _drain0():
                        wait_scatter(nb)
                else:
                    wait_scatter(nb)
                start_gather(g + 2, nb)
                wait_gather(b)
                scale(g, b)
                start_scatter(g, b)
            return carry

        lax.fori_loop(0, TRI, tri_body, 0)
        wait_scatter(2)
        wait_gather(0)
        scale(G_STAGE - 2, 0)
        start_scatter(G_STAGE - 2, 0)
        wait_gather(1)
        scale(G_STAGE - 1, 1)
        start_scatter(G_STAGE - 1, 1)
        wait_scatter(0)
        wait_scatter(1)

    plsc.subcore_barrier()
    for hs in range(N_STAGE):
        run_stage(hs)

    plsc.subcore_barrier()
    pltpu.sync_copy(acc.at[pl.ds(s * ROWS_PER_SUB, ROWS_PER_SUB)],
                    out_hbm.at[c, pl.ds(s * ROWS_PER_SUB, ROWS_PER_SUB)])

    @pl.when(s == NUM_SUBCORES - 1)
    def _write_tail():
        pltpu.sync_copy(acc.at[pl.ds(ROW_TAIL_OFF, ROW_TAIL)],
                        out_hbm.at[c, pl.ds(ROW_TAIL_OFF, ROW_TAIL)])


_sc_scatter = functools.partial(
    pl.kernel,
    out_type=jax.ShapeDtypeStruct((NUM_CORES, N, H), jnp.float32),
    mesh=plsc.VectorSubcoreMesh(core_axis_name="c", subcore_axis_name="s"),
    scratch_types=[
        pltpu.VMEM_SHARED((N, H), jnp.float32),
        pltpu.VMEM((G_STAGE, GROUP), jnp.int32),
        pltpu.VMEM((G_STAGE, GROUP), jnp.int32),
        pltpu.VMEM((G_STAGE, GROUP), jnp.float32),
        pltpu.VMEM((NBUF, GROUP, H), jnp.float32),
        pltpu.SemaphoreType.DMA,
        pltpu.SemaphoreType.DMA,
        pltpu.SemaphoreType.DMA,
        pltpu.SemaphoreType.DMA,
        pltpu.SemaphoreType.DMA,
        pltpu.SemaphoreType.DMA,
    ],
)(_sc_body)


def _mm(a, b):
    return jax.lax.dot_general(a, b, (((1,), (0,)), ((), ())),
                               preferred_element_type=jnp.float32)


def _tc_in_body(x_ref, win_ref, bin_ref, wm_ref, bm_ref, h_ref, m_ref):
    h = jnp.maximum(_mm(x_ref[...], win_ref[...]) + bin_ref[...], 0.0)
    h_ref[...] = h
    m_ref[...] = _mm(h, wm_ref[...]) + bm_ref[...]


def _tc_mid_body(h_ref, p0_ref, p1_ref, wu_ref, bu_ref, wm_ref, bm_ref,
                 h2_ref, m2_ref):
    t = h_ref[...] + p0_ref[...] + p1_ref[...]
    h2 = jnp.maximum(_mm(t, wu_ref[...]) + bu_ref[...], 0.0)
    h2_ref[...] = h2
    m2_ref[...] = _mm(h2, wm_ref[...]) + bm_ref[...]


def _tc_out_body(h_ref, p0_ref, p1_ref, wu_ref, bu_ref, who_ref, bho_ref,
                 z_ref):
    t = h_ref[...] + p0_ref[...] + p1_ref[...]
    h3 = jnp.maximum(_mm(t, wu_ref[...]) + bu_ref[...], 0.0)
    z = _mm(h3, who_ref[...]) + bho_ref[...]
    col = lax.broadcasted_iota(jnp.int32, z.shape, 1)
    z_ref[...] = jnp.where(col == 0, jax.nn.sigmoid(z), jnp.tanh(z))


_row_spec = pl.BlockSpec((_TC_BLOCK, H), lambda i: (i, 0))
_FULL_W = pl.BlockSpec((H, H), lambda i: (0, 0))
_FULL_B = pl.BlockSpec((1, H), lambda i: (0, 0))
_GRID = (N // _TC_BLOCK,)

_tc_in = pl.pallas_call(
    _tc_in_body,
    grid=_GRID,
    in_specs=[_row_spec, _FULL_W, _FULL_B, _FULL_W, _FULL_B],
    out_specs=[_row_spec, _row_spec],
    out_shape=[jax.ShapeDtypeStruct((N, H), jnp.float32),
               jax.ShapeDtypeStruct((N, H), jnp.float32)],
)

_tc_mid = pl.pallas_call(
    _tc_mid_body,
    grid=_GRID,
    in_specs=[_row_spec] * 3 + [_FULL_W, _FULL_B, _FULL_W, _FULL_B],
    out_specs=[_row_spec, _row_spec],
    out_shape=[jax.ShapeDtypeStruct((N, H), jnp.float32),
               jax.ShapeDtypeStruct((N, H), jnp.float32)],
)

_tc_out = pl.pallas_call(
    _tc_out_body,
    grid=_GRID,
    in_specs=[_row_spec] * 3 + [_FULL_W, _FULL_B, _FULL_W, _FULL_B],
    out_specs=[_row_spec],
    out_shape=[jax.ShapeDtypeStruct((N, H), jnp.float32)],
)


def kernel(x, src, dst, w, Win, b_in, Wm, bm, Wu, bu, Wos, bos, Wside, bside):
    pad = EP - E
    srcp = jnp.concatenate([src.astype(jnp.int32),
                            jnp.zeros((pad,), jnp.int32)]).reshape(-1, GROUP)
    dstp = jnp.concatenate([dst.astype(jnp.int32),
                            jnp.zeros((pad,), jnp.int32)]).reshape(-1, GROUP)
    wp = jnp.concatenate([w, jnp.full((pad,), -1e30, jnp.float32)]
                         ).reshape(-1, GROUP)
    zero_rows = jnp.zeros((ROWS_PER_SUB, H), jnp.float32)

    bin2 = b_in.reshape(1, H)
    who = jnp.zeros((H, H), jnp.float32)
    who = who.at[:, 0].set(Wos[:, 0]).at[:, 1].set(Wside[:, 0])
    bho = jnp.zeros((H,), jnp.float32).at[0].set(bos[0]).at[1].set(bside[0])
    bho = bho.reshape(1, H)

    h, m = _tc_in(x, Win, bin2, Wm[0], bm[0].reshape(1, H))

    for k in range(L):
        partials = _sc_scatter(m, srcp, dstp, wp, zero_rows)
        p0, p1 = partials[0], partials[1]
        if k < L - 1:
            h, m = _tc_mid(h, p0, p1, Wu[k], bu[k].reshape(1, H),
                           Wm[k + 1], bm[k + 1].reshape(1, H))
        else:
            z, = _tc_out(h, p0, p1, Wu[k], bu[k].reshape(1, H), who, bho)

    return (z[:, 0], z[:, 1])


# speedup vs baseline: 3.8743x; 1.1845x over previous
import functools

import jax
import jax.numpy as jnp
from jax import lax
from jax.experimental import pallas as pl
from jax.experimental.pallas import tpu as pltpu
from jax.experimental.pallas import tpu_sc as plsc

N = 10000
E = 320000
D = 128
H = 128
L = 3

NUM_CORES = 2
NUM_SUBCORES = 16
NUM_WORKERS = NUM_CORES * NUM_SUBCORES
GROUP = 64
G_STAGE = 32
G_PER_PAIR = 320
FAST_CORE = 0
FAST_STAGES = 8
SLOW_STAGES = G_PER_PAIR // G_STAGE - FAST_STAGES
NBUF = 3
TRI = (G_STAGE - 2) // NBUF
EP = NUM_SUBCORES * G_PER_PAIR * GROUP
ROWS_PER_SUB = 624
ROW_TAIL = N - NUM_SUBCORES * ROWS_PER_SUB
ROW_TAIL_OFF = NUM_SUBCORES * ROWS_PER_SUB

_TC_BLOCK = 1000


def _sc_body(m_hbm, src_hbm, dst_hbm, w_hbm, zero_hbm,
             out_hbm, acc, src_v, dst_v, att_v, rows_v,
             gs0, gs1, gs2, ss0, ss1, ss2):
    gsems = (gs0, gs1, gs2)
    ssems = (ss0, ss1, ss2)
    c = lax.axis_index("c")
    s = lax.axis_index("s")

    pltpu.sync_copy(zero_hbm.at[pl.ds(0, ROWS_PER_SUB)],
                    acc.at[pl.ds(s * ROWS_PER_SUB, ROWS_PER_SUB)])

    @pl.when(s == NUM_SUBCORES - 1)
    def _zero_tail():
        pltpu.sync_copy(zero_hbm.at[pl.ds(0, ROW_TAIL)],
                        acc.at[pl.ds(ROW_TAIL_OFF, ROW_TAIL)])

    def start_gather(g, b):
        pltpu.async_copy(m_hbm.at[src_v.at[g]], rows_v.at[b], gsems[b])

    def wait_gather(b):
        pltpu.make_async_copy(m_hbm.at[src_v.at[0]], rows_v.at[b],
                              gsems[b]).wait()

    def start_scatter(g, b):
        pltpu.async_copy(rows_v.at[b], acc.at[dst_v.at[g]], ssems[b],
                         add=True)

    def wait_scatter(b):
        pltpu.make_async_copy(rows_v.at[b], acc.at[dst_v.at[0]],
                              ssems[b]).wait()

    def scale(g, b):
        def edge_chunk_body(cidx, inner):
            e0 = cidx * 16
            av = att_v[g, pl.ds(e0, 16)]
            for j in range(16):
                a = av[j]
                for f in range(H // 16):
                    sl = pl.ds(f * 16, 16)
                    rows_v[b, e0 + j, sl] = rows_v[b, e0 + j, sl] * a
            return inner

        lax.fori_loop(0, GROUP // 16, edge_chunk_body, 0)

    def run_stage(base):
        pltpu.sync_copy(src_hbm.at[pl.ds(base, G_STAGE)], src_v)
        pltpu.sync_copy(dst_hbm.at[pl.ds(base, G_STAGE)], dst_v)
        pltpu.sync_copy(w_hbm.at[pl.ds(base, G_STAGE)], att_v)

        n_sl = GROUP // 16

        def sig_body(j, carry):
            g = j // n_sl
            f = (j % n_sl) * 16
            wv = att_v[g, pl.ds(f, 16)]
            att_v[g, pl.ds(f, 16)] = 1.0 / (1.0 + jnp.exp(-wv))
            return carry

        lax.fori_loop(0, G_STAGE * n_sl, sig_body, 0)

        start_gather(0, 0)
        start_gather(1, 1)

        def tri_body(i, carry):
            for b in range(NBUF):
                g = 3 * i + b
                nb = (b + 2) % 3
                if b == 0:
                    @pl.when(i > 0)
                    def _drain0():
                        wait_scatter(nb)
                else:
                    wait_scatter(nb)
                start_gather(g + 2, nb)
                wait_gather(b)
                scale(g, b)
                start_scatter(g, b)
            return carry

        lax.fori_loop(0, TRI, tri_body, 0)
        wait_scatter(2)
        wait_gather(0)
        scale(G_STAGE - 2, 0)
        start_scatter(G_STAGE - 2, 0)
        wait_gather(1)
        scale(G_STAGE - 1, 1)
        start_scatter(G_STAGE - 1, 1)
        wait_scatter(0)
        wait_scatter(1)

    plsc.subcore_barrier()
    n_stages = lax.select(c == FAST_CORE, FAST_STAGES, SLOW_STAGES)
    pair_off = lax.select(c == FAST_CORE, 0, FAST_STAGES * G_STAGE)

    def stage_body(hs, carry):
        run_stage(s * G_PER_PAIR + pair_off + hs * G_STAGE)
        return carry

    lax.fori_loop(0, n_stages, stage_body, 0)

    plsc.subcore_barrier()
    pltpu.sync_copy(acc.at[pl.ds(s * ROWS_PER_SUB, ROWS_PER_SUB)],
                    out_hbm.at[c, pl.ds(s * ROWS_PER_SUB, ROWS_PER_SUB)])

    @pl.when(s == NUM_SUBCORES - 1)
    def _write_tail():
        pltpu.sync_copy(acc.at[pl.ds(ROW_TAIL_OFF, ROW_TAIL)],
                        out_hbm.at[c, pl.ds(ROW_TAIL_OFF, ROW_TAIL)])


_sc_scatter = functools.partial(
    pl.kernel,
    out_type=jax.ShapeDtypeStruct((NUM_CORES, N, H), jnp.float32),
    mesh=plsc.VectorSubcoreMesh(core_axis_name="c", subcore_axis_name="s"),
    scratch_types=[
        pltpu.VMEM_SHARED((N, H), jnp.float32),
        pltpu.VMEM((G_STAGE, GROUP), jnp.int32),
        pltpu.VMEM((G_STAGE, GROUP), jnp.int32),
        pltpu.VMEM((G_STAGE, GROUP), jnp.float32),
        pltpu.VMEM((NBUF, GROUP, H), jnp.float32),
        pltpu.SemaphoreType.DMA,
        pltpu.SemaphoreType.DMA,
        pltpu.SemaphoreType.DMA,
        pltpu.SemaphoreType.DMA,
        pltpu.SemaphoreType.DMA,
        pltpu.SemaphoreType.DMA,
    ],
)(_sc_body)


def _mm(a, b):
    return jax.lax.dot_general(a, b, (((1,), (0,)), ((), ())),
                               preferred_element_type=jnp.float32)


def _tc_in_body(x_ref, win_ref, bin_ref, wm_ref, bm_ref, h_ref, m_ref):
    h = jnp.maximum(_mm(x_ref[...], win_ref[...]) + bin_ref[...], 0.0)
    h_ref[...] = h
    m_ref[...] = _mm(h, wm_ref[...]) + bm_ref[...]


def _tc_mid_body(h_ref, p0_ref, p1_ref, wu_ref, bu_ref, wm_ref, bm_ref,
                 h2_ref, m2_ref):
    t = h_ref[...] + p0_ref[...] + p1_ref[...]
    h2 = jnp.maximum(_mm(t, wu_ref[...]) + bu_ref[...], 0.0)
    h2_ref[...] = h2
    m2_ref[...] = _mm(h2, wm_ref[...]) + bm_ref[...]


def _tc_out_body(h_ref, p0_ref, p1_ref, wu_ref, bu_ref, who_ref, bho_ref,
                 z_ref):
    t = h_ref[...] + p0_ref[...] + p1_ref[...]
    h3 = jnp.maximum(_mm(t, wu_ref[...]) + bu_ref[...], 0.0)
    z = _mm(h3, who_ref[...]) + bho_ref[...]
    col = lax.broadcasted_iota(jnp.int32, z.shape, 1)
    z_ref[...] = jnp.where(col == 0, jax.nn.sigmoid(z), jnp.tanh(z))


_row_spec = pl.BlockSpec((_TC_BLOCK, H), lambda i: (i, 0))
_FULL_W = pl.BlockSpec((H, H), lambda i: (0, 0))
_FULL_B = pl.BlockSpec((1, H), lambda i: (0, 0))
_GRID = (N // _TC_BLOCK,)

_tc_in = pl.pallas_call(
    _tc_in_body,
    grid=_GRID,
    in_specs=[_row_spec, _FULL_W, _FULL_B, _FULL_W, _FULL_B],
    out_specs=[_row_spec, _row_spec],
    out_shape=[jax.ShapeDtypeStruct((N, H), jnp.float32),
               jax.ShapeDtypeStruct((N, H), jnp.float32)],
)

_tc_mid = pl.pallas_call(
    _tc_mid_body,
    grid=_GRID,
    in_specs=[_row_spec] * 3 + [_FULL_W, _FULL_B, _FULL_W, _FULL_B],
    out_specs=[_row_spec, _row_spec],
    out_shape=[jax.ShapeDtypeStruct((N, H), jnp.float32),
               jax.ShapeDtypeStruct((N, H), jnp.float32)],
)

_tc_out = pl.pallas_call(
    _tc_out_body,
    grid=_GRID,
    in_specs=[_row_spec] * 3 + [_FULL_W, _FULL_B, _FULL_W, _FULL_B],
    out_specs=[_row_spec],
    out_shape=[jax.ShapeDtypeStruct((N, H), jnp.float32)],
)


def kernel(x, src, dst, w, Win, b_in, Wm, bm, Wu, bu, Wos, bos, Wside, bside):
    pad = EP - E
    srcp = jnp.concatenate([src.astype(jnp.int32),
                            jnp.zeros((pad,), jnp.int32)]).reshape(-1, GROUP)
    dstp = jnp.concatenate([dst.astype(jnp.int32),
                            jnp.zeros((pad,), jnp.int32)]).reshape(-1, GROUP)
    wp = jnp.concatenate([w, jnp.full((pad,), -1e30, jnp.float32)]
                         ).reshape(-1, GROUP)
    zero_rows = jnp.zeros((ROWS_PER_SUB, H), jnp.float32)

    bin2 = b_in.reshape(1, H)
    who = jnp.zeros((H, H), jnp.float32)
    who = who.at[:, 0].set(Wos[:, 0]).at[:, 1].set(Wside[:, 0])
    bho = jnp.zeros((H,), jnp.float32).at[0].set(bos[0]).at[1].set(bside[0])
    bho = bho.reshape(1, H)

    h, m = _tc_in(x, Win, bin2, Wm[0], bm[0].reshape(1, H))

    for k in range(L):
        partials = _sc_scatter(m, srcp, dstp, wp, zero_rows)
        p0, p1 = partials[0], partials[1]
        if k < L - 1:
            h, m = _tc_mid(h, p0, p1, Wu[k], bu[k].reshape(1, H),
                           Wm[k + 1], bm[k + 1].reshape(1, H))
        else:
            z, = _tc_out(h, p0, p1, Wu[k], bu[k].reshape(1, H), who, bho)

    return (z[:, 0], z[:, 1])


# speedup vs baseline: 3.9638x; 1.0231x over previous
import functools

import jax
import jax.numpy as jnp
from jax import lax
from jax.experimental import pallas as pl
from jax.experimental.pallas import tpu as pltpu
from jax.experimental.pallas import tpu_sc as plsc

N = 10000
E = 320000
D = 128
H = 128
L = 3

NUM_CORES = 2
NUM_SUBCORES = 16
NUM_WORKERS = NUM_CORES * NUM_SUBCORES
GROUP = 64
G_STAGE = 32
G_PER_PAIR = 320
FAST_CORE = 0
FAST_STAGES = 7
SLOW_STAGES = G_PER_PAIR // G_STAGE - FAST_STAGES
NBUF = 3
TRI = (G_STAGE - 2) // NBUF
EP = NUM_SUBCORES * G_PER_PAIR * GROUP
ROWS_PER_SUB = 624
ROW_TAIL = N - NUM_SUBCORES * ROWS_PER_SUB
ROW_TAIL_OFF = NUM_SUBCORES * ROWS_PER_SUB

_TC_BLOCK = 1000


def _sc_body(m_hbm, src_hbm, dst_hbm, w_hbm, zero_hbm,
             out_hbm, acc, src_v, dst_v, att_v, rows_v,
             gs0, gs1, gs2, ss0, ss1, ss2):
    gsems = (gs0, gs1, gs2)
    ssems = (ss0, ss1, ss2)
    c = lax.axis_index("c")
    s = lax.axis_index("s")

    pltpu.sync_copy(zero_hbm.at[pl.ds(0, ROWS_PER_SUB)],
                    acc.at[pl.ds(s * ROWS_PER_SUB, ROWS_PER_SUB)])

    @pl.when(s == NUM_SUBCORES - 1)
    def _zero_tail():
        pltpu.sync_copy(zero_hbm.at[pl.ds(0, ROW_TAIL)],
                        acc.at[pl.ds(ROW_TAIL_OFF, ROW_TAIL)])

    def start_gather(g, b):
        pltpu.async_copy(m_hbm.at[src_v.at[g]], rows_v.at[b], gsems[b])

    def wait_gather(b):
        pltpu.make_async_copy(m_hbm.at[src_v.at[0]], rows_v.at[b],
                              gsems[b]).wait()

    def start_scatter(g, b):
        pltpu.async_copy(rows_v.at[b], acc.at[dst_v.at[g]], ssems[b],
                         add=True)

    def wait_scatter(b):
        pltpu.make_async_copy(rows_v.at[b], acc.at[dst_v.at[0]],
                              ssems[b]).wait()

    def scale(g, b):
        def edge_chunk_body(cidx, inner):
            e0 = cidx * 16
            av = att_v[g, pl.ds(e0, 16)]
            for j in range(16):
                a = av[j]
                for f in range(H // 16):
                    sl = pl.ds(f * 16, 16)
                    rows_v[b, e0 + j, sl] = rows_v[b, e0 + j, sl] * a
            return inner

        lax.fori_loop(0, GROUP // 16, edge_chunk_body, 0)

    def run_stage(base):
        pltpu.sync_copy(src_hbm.at[pl.ds(base, G_STAGE)], src_v)
        pltpu.sync_copy(dst_hbm.at[pl.ds(base, G_STAGE)], dst_v)
        pltpu.sync_copy(w_hbm.at[pl.ds(base, G_STAGE)], att_v)

        n_sl = GROUP // 16

        def sig_body(j, carry):
            g = j // n_sl
            f = (j % n_sl) * 16
            wv = att_v[g, pl.ds(f, 16)]
            att_v[g, pl.ds(f, 16)] = 1.0 / (1.0 + jnp.exp(-wv))
            return carry

        lax.fori_loop(0, G_STAGE * n_sl, sig_body, 0)

        start_gather(0, 0)
        start_gather(1, 1)

        def tri_body(i, carry):
            for b in range(NBUF):
                g = 3 * i + b
                nb = (b + 2) % 3
                if b == 0:
                    @pl.when(i > 0)
                    def _drain0():
                        wait_scatter(nb)
                else:
                    wait_scatter(nb)
                start_gather(g + 2, nb)
                wait_gather(b)
                scale(g, b)
                start_scatter(g, b)
            return carry

        lax.fori_loop(0, TRI, tri_body, 0)
        wait_scatter(2)
        wait_gather(0)
        scale(G_STAGE - 2, 0)
        start_scatter(G_STAGE - 2, 0)
        wait_gather(1)
        scale(G_STAGE - 1, 1)
        start_scatter(G_STAGE - 1, 1)
        wait_scatter(0)
        wait_scatter(1)

    plsc.subcore_barrier()
    n_stages = lax.select(c == FAST_CORE, FAST_STAGES, SLOW_STAGES)
    pair_off = lax.select(c == FAST_CORE, 0, FAST_STAGES * G_STAGE)

    def stage_body(hs, carry):
        run_stage(s * G_PER_PAIR + pair_off + hs * G_STAGE)
        return carry

    lax.fori_loop(0, n_stages, stage_body, 0)

    plsc.subcore_barrier()
    pltpu.sync_copy(acc.at[pl.ds(s * ROWS_PER_SUB, ROWS_PER_SUB)],
                    out_hbm.at[c, pl.ds(s * ROWS_PER_SUB, ROWS_PER_SUB)])

    @pl.when(s == NUM_SUBCORES - 1)
    def _write_tail():
        pltpu.sync_copy(acc.at[pl.ds(ROW_TAIL_OFF, ROW_TAIL)],
                        out_hbm.at[c, pl.ds(ROW_TAIL_OFF, ROW_TAIL)])


_sc_scatter = functools.partial(
    pl.kernel,
    out_type=jax.ShapeDtypeStruct((NUM_CORES, N, H), jnp.float32),
    mesh=plsc.VectorSubcoreMesh(core_axis_name="c", subcore_axis_name="s"),
    scratch_types=[
        pltpu.VMEM_SHARED((N, H), jnp.float32),
        pltpu.VMEM((G_STAGE, GROUP), jnp.int32),
        pltpu.VMEM((G_STAGE, GROUP), jnp.int32),
        pltpu.VMEM((G_STAGE, GROUP), jnp.float32),
        pltpu.VMEM((NBUF, GROUP, H), jnp.float32),
        pltpu.SemaphoreType.DMA,
        pltpu.SemaphoreType.DMA,
        pltpu.SemaphoreType.DMA,
        pltpu.SemaphoreType.DMA,
        pltpu.SemaphoreType.DMA,
        pltpu.SemaphoreType.DMA,
    ],
)(_sc_body)


def _mm(a, b):
    return jax.lax.dot_general(a, b, (((1,), (0,)), ((), ())),
                               preferred_element_type=jnp.float32)


def _tc_in_body(x_ref, win_ref, bin_ref, wm_ref, bm_ref, h_ref, m_ref):
    h = jnp.maximum(_mm(x_ref[...], win_ref[...]) + bin_ref[...], 0.0)
    h_ref[...] = h
    m_ref[...] = _mm(h, wm_ref[...]) + bm_ref[...]


def _tc_mid_body(h_ref, p0_ref, p1_ref, wu_ref, bu_ref, wm_ref, bm_ref,
                 h2_ref, m2_ref):
    t = h_ref[...] + p0_ref[...] + p1_ref[...]
    h2 = jnp.maximum(_mm(t, wu_ref[...]) + bu_ref[...], 0.0)
    h2_ref[...] = h2
    m2_ref[...] = _mm(h2, wm_ref[...]) + bm_ref[...]


def _tc_out_body(h_ref, p0_ref, p1_ref, wu_ref, bu_ref, who_ref, bho_ref,
                 z_ref):
    t = h_ref[...] + p0_ref[...] + p1_ref[...]
    h3 = jnp.maximum(_mm(t, wu_ref[...]) + bu_ref[...], 0.0)
    z = _mm(h3, who_ref[...]) + bho_ref[...]
    col = lax.broadcasted_iota(jnp.int32, z.shape, 1)
    z_ref[...] = jnp.where(col == 0, jax.nn.sigmoid(z), jnp.tanh(z))


_row_spec = pl.BlockSpec((_TC_BLOCK, H), lambda i: (i, 0))
_FULL_W = pl.BlockSpec((H, H), lambda i: (0, 0))
_FULL_B = pl.BlockSpec((1, H), lambda i: (0, 0))
_GRID = (N // _TC_BLOCK,)

_tc_in = pl.pallas_call(
    _tc_in_body,
    grid=_GRID,
    in_specs=[_row_spec, _FULL_W, _FULL_B, _FULL_W, _FULL_B],
    out_specs=[_row_spec, _row_spec],
    out_shape=[jax.ShapeDtypeStruct((N, H), jnp.float32),
               jax.ShapeDtypeStruct((N, H), jnp.float32)],
)

_tc_mid = pl.pallas_call(
    _tc_mid_body,
    grid=_GRID,
    in_specs=[_row_spec] * 3 + [_FULL_W, _FULL_B, _FULL_W, _FULL_B],
    out_specs=[_row_spec, _row_spec],
    out_shape=[jax.ShapeDtypeStruct((N, H), jnp.float32),
               jax.ShapeDtypeStruct((N, H), jnp.float32)],
)

_tc_out = pl.pallas_call(
    _tc_out_body,
    grid=_GRID,
    in_specs=[_row_spec] * 3 + [_FULL_W, _FULL_B, _FULL_W, _FULL_B],
    out_specs=[_row_spec],
    out_shape=[jax.ShapeDtypeStruct((N, H), jnp.float32)],
)


def kernel(x, src, dst, w, Win, b_in, Wm, bm, Wu, bu, Wos, bos, Wside, bside):
    pad = EP - E
    srcp = jnp.concatenate([src.astype(jnp.int32),
                            jnp.zeros((pad,), jnp.int32)]).reshape(-1, GROUP)
    dstp = jnp.concatenate([dst.astype(jnp.int32),
                            jnp.zeros((pad,), jnp.int32)]).reshape(-1, GROUP)
    wp = jnp.concatenate([w, jnp.full((pad,), -1e30, jnp.float32)]
                         ).reshape(-1, GROUP)
    zero_rows = jnp.zeros((ROWS_PER_SUB, H), jnp.float32)

    bin2 = b_in.reshape(1, H)
    who = jnp.zeros((H, H), jnp.float32)
    who = who.at[:, 0].set(Wos[:, 0]).at[:, 1].set(Wside[:, 0])
    bho = jnp.zeros((H,), jnp.float32).at[0].set(bos[0]).at[1].set(bside[0])
    bho = bho.reshape(1, H)

    h, m = _tc_in(x, Win, bin2, Wm[0], bm[0].reshape(1, H))

    for k in range(L):
        partials = _sc_scatter(m, srcp, dstp, wp, zero_rows)
        p0, p1 = partials[0], partials[1]
        if k < L - 1:
            h, m = _tc_mid(h, p0, p1, Wu[k], bu[k].reshape(1, H),
                           Wm[k + 1], bm[k + 1].reshape(1, H))
        else:
            z, = _tc_out(h, p0, p1, Wu[k], bu[k].reshape(1, H), who, bho)

    return (z[:, 0], z[:, 1])


# speedup vs baseline: 4.0190x; 1.0139x over previous
import functools

import jax
import jax.numpy as jnp
from jax import lax
from jax.experimental import pallas as pl
from jax.experimental.pallas import tpu as pltpu
from jax.experimental.pallas import tpu_sc as plsc

N = 10000
E = 320000
D = 128
H = 128
L = 3

NUM_CORES = 2
NUM_SUBCORES = 16
NUM_WORKERS = NUM_CORES * NUM_SUBCORES
GROUP = 64
G_STAGE = 32
G_SMALL = 8
G_PER_PAIR = 320
FAST_CORE = 0
FAST_GROUPS = 7 * G_STAGE + G_SMALL
NBUF = 3
EP = NUM_SUBCORES * G_PER_PAIR * GROUP
ROWS_PER_SUB = 624
ROW_TAIL = N - NUM_SUBCORES * ROWS_PER_SUB
ROW_TAIL_OFF = NUM_SUBCORES * ROWS_PER_SUB

_TC_BLOCK = 1000


def _sc_body(m_hbm, src_hbm, dst_hbm, w_hbm, zero_hbm,
             out_hbm, acc, src_v, dst_v, att_v, rows_v,
             gs0, gs1, gs2, ss0, ss1, ss2):
    gsems = (gs0, gs1, gs2)
    ssems = (ss0, ss1, ss2)
    c = lax.axis_index("c")
    s = lax.axis_index("s")

    pltpu.sync_copy(zero_hbm.at[pl.ds(0, ROWS_PER_SUB)],
                    acc.at[pl.ds(s * ROWS_PER_SUB, ROWS_PER_SUB)])

    @pl.when(s == NUM_SUBCORES - 1)
    def _zero_tail():
        pltpu.sync_copy(zero_hbm.at[pl.ds(0, ROW_TAIL)],
                        acc.at[pl.ds(ROW_TAIL_OFF, ROW_TAIL)])

    def start_gather(g, b):
        pltpu.async_copy(m_hbm.at[src_v.at[g]], rows_v.at[b], gsems[b])

    def wait_gather(b):
        pltpu.make_async_copy(m_hbm.at[src_v.at[0]], rows_v.at[b],
                              gsems[b]).wait()

    def start_scatter(g, b):
        pltpu.async_copy(rows_v.at[b], acc.at[dst_v.at[g]], ssems[b],
                         add=True)

    def wait_scatter(b):
        pltpu.make_async_copy(rows_v.at[b], acc.at[dst_v.at[0]],
                              ssems[b]).wait()

    def scale(g, b):
        def edge_chunk_body(cidx, inner):
            e0 = cidx * 16
            av = att_v[g, pl.ds(e0, 16)]
            for j in range(16):
                a = av[j]
                for f in range(H // 16):
                    sl = pl.ds(f * 16, 16)
                    rows_v[b, e0 + j, sl] = rows_v[b, e0 + j, sl] * a
            return inner

        lax.fori_loop(0, GROUP // 16, edge_chunk_body, 0)

    def run_stage(base, ng):
        pltpu.sync_copy(src_hbm.at[pl.ds(base, ng)],
                        src_v.at[pl.ds(0, ng)])
        pltpu.sync_copy(dst_hbm.at[pl.ds(base, ng)],
                        dst_v.at[pl.ds(0, ng)])
        pltpu.sync_copy(w_hbm.at[pl.ds(base, ng)],
                        att_v.at[pl.ds(0, ng)])

        n_sl = GROUP // 16

        def sig_body(j, carry):
            g = j // n_sl
            f = (j % n_sl) * 16
            wv = att_v[g, pl.ds(f, 16)]
            att_v[g, pl.ds(f, 16)] = 1.0 / (1.0 + jnp.exp(-wv))
            return carry

        lax.fori_loop(0, ng * n_sl, sig_body, 0)

        start_gather(0, 0)
        start_gather(1, 1)

        def tri_body(i, carry):
            for b in range(NBUF):
                g = 3 * i + b
                nb = (b + 2) % 3
                if b == 0:
                    @pl.when(i > 0)
                    def _drain0():
                        wait_scatter(nb)
                else:
                    wait_scatter(nb)
                start_gather(g + 2, nb)
                wait_gather(b)
                scale(g, b)
                start_scatter(g, b)
            return carry

        lax.fori_loop(0, (ng - 2) // NBUF, tri_body, 0)
        wait_scatter(2)
        wait_gather(0)
        scale(ng - 2, 0)
        start_scatter(ng - 2, 0)
        wait_gather(1)
        scale(ng - 1, 1)
        start_scatter(ng - 1, 1)
        wait_scatter(0)
        wait_scatter(1)

    plsc.subcore_barrier()

    @pl.when(c == FAST_CORE)
    def _fast_sched():
        def body(hs, carry):
            run_stage(s * G_PER_PAIR + hs * G_STAGE, G_STAGE)
            return carry

        lax.fori_loop(0, 7, body, 0)
        run_stage(s * G_PER_PAIR + 7 * G_STAGE, G_SMALL)

    @pl.when(c != FAST_CORE)
    def _slow_sched():
        def body(hs, carry):
            run_stage(s * G_PER_PAIR + FAST_GROUPS + hs * G_STAGE, G_STAGE)
            return carry

        lax.fori_loop(0, 2, body, 0)

        def body2(hs, carry):
            run_stage(s * G_PER_PAIR + FAST_GROUPS + 2 * G_STAGE
                      + hs * G_SMALL, G_SMALL)
            return carry

        lax.fori_loop(0, 3, body2, 0)

    plsc.subcore_barrier()
    pltpu.sync_copy(acc.at[pl.ds(s * ROWS_PER_SUB, ROWS_PER_SUB)],
                    out_hbm.at[c, pl.ds(s * ROWS_PER_SUB, ROWS_PER_SUB)])

    @pl.when(s == NUM_SUBCORES - 1)
    def _write_tail():
        pltpu.sync_copy(acc.at[pl.ds(ROW_TAIL_OFF, ROW_TAIL)],
                        out_hbm.at[c, pl.ds(ROW_TAIL_OFF, ROW_TAIL)])


_sc_scatter = functools.partial(
    pl.kernel,
    out_type=jax.ShapeDtypeStruct((NUM_CORES, N, H), jnp.float32),
    mesh=plsc.VectorSubcoreMesh(core_axis_name="c", subcore_axis_name="s"),
    scratch_types=[
        pltpu.VMEM_SHARED((N, H), jnp.float32),
        pltpu.VMEM((G_STAGE, GROUP), jnp.int32),
        pltpu.VMEM((G_STAGE, GROUP), jnp.int32),
        pltpu.VMEM((G_STAGE, GROUP), jnp.float32),
        pltpu.VMEM((NBUF, GROUP, H), jnp.float32),
        pltpu.SemaphoreType.DMA,
        pltpu.SemaphoreType.DMA,
        pltpu.SemaphoreType.DMA,
        pltpu.SemaphoreType.DMA,
        pltpu.SemaphoreType.DMA,
        pltpu.SemaphoreType.DMA,
    ],
)(_sc_body)


def _mm(a, b):
    return jax.lax.dot_general(a, b, (((1,), (0,)), ((), ())),
                               preferred_element_type=jnp.float32)


def _tc_in_body(x_ref, win_ref, bin_ref, wm_ref, bm_ref, h_ref, m_ref):
    h = jnp.maximum(_mm(x_ref[...], win_ref[...]) + bin_ref[...], 0.0)
    h_ref[...] = h
    m_ref[...] = _mm(h, wm_ref[...]) + bm_ref[...]


def _tc_mid_body(h_ref, p0_ref, p1_ref, wu_ref, bu_ref, wm_ref, bm_ref,
                 h2_ref, m2_ref):
    t = h_ref[...] + p0_ref[...] + p1_ref[...]
    h2 = jnp.maximum(_mm(t, wu_ref[...]) + bu_ref[...], 0.0)
    h2_ref[...] = h2
    m2_ref[...] = _mm(h2, wm_ref[...]) + bm_ref[...]


def _tc_out_body(h_ref, p0_ref, p1_ref, wu_ref, bu_ref, who_ref, bho_ref,
                 z_ref):
    t = h_ref[...] + p0_ref[...] + p1_ref[...]
    h3 = jnp.maximum(_mm(t, wu_ref[...]) + bu_ref[...], 0.0)
    z = _mm(h3, who_ref[...]) + bho_ref[...]
    col = lax.broadcasted_iota(jnp.int32, z.shape, 1)
    z_ref[...] = jnp.where(col == 0, jax.nn.sigmoid(z), jnp.tanh(z))


_row_spec = pl.BlockSpec((_TC_BLOCK, H), lambda i: (i, 0))
_FULL_W = pl.BlockSpec((H, H), lambda i: (0, 0))
_FULL_B = pl.BlockSpec((1, H), lambda i: (0, 0))
_GRID = (N // _TC_BLOCK,)

_tc_in = pl.pallas_call(
    _tc_in_body,
    grid=_GRID,
    in_specs=[_row_spec, _FULL_W, _FULL_B, _FULL_W, _FULL_B],
    out_specs=[_row_spec, _row_spec],
    out_shape=[jax.ShapeDtypeStruct((N, H), jnp.float32),
               jax.ShapeDtypeStruct((N, H), jnp.float32)],
)

_tc_mid = pl.pallas_call(
    _tc_mid_body,
    grid=_GRID,
    in_specs=[_row_spec] * 3 + [_FULL_W, _FULL_B, _FULL_W, _FULL_B],
    out_specs=[_row_spec, _row_spec],
    out_shape=[jax.ShapeDtypeStruct((N, H), jnp.float32),
               jax.ShapeDtypeStruct((N, H), jnp.float32)],
)

_tc_out = pl.pallas_call(
    _tc_out_body,
    grid=_GRID,
    in_specs=[_row_spec] * 3 + [_FULL_W, _FULL_B, _FULL_W, _FULL_B],
    out_specs=[_row_spec],
    out_shape=[jax.ShapeDtypeStruct((N, H), jnp.float32)],
)


def kernel(x, src, dst, w, Win, b_in, Wm, bm, Wu, bu, Wos, bos, Wside, bside):
    pad = EP - E
    srcp = jnp.concatenate([src.astype(jnp.int32),
                            jnp.zeros((pad,), jnp.int32)]).reshape(-1, GROUP)
    dstp = jnp.concatenate([dst.astype(jnp.int32),
                            jnp.zeros((pad,), jnp.int32)]).reshape(-1, GROUP)
    wp = jnp.concatenate([w, jnp.full((pad,), -1e30, jnp.float32)]
                         ).reshape(-1, GROUP)
    zero_rows = jnp.zeros((ROWS_PER_SUB, H), jnp.float32)

    bin2 = b_in.reshape(1, H)
    who = jnp.zeros((H, H), jnp.float32)
    who = who.at[:, 0].set(Wos[:, 0]).at[:, 1].set(Wside[:, 0])
    bho = jnp.zeros((H,), jnp.float32).at[0].set(bos[0]).at[1].set(bside[0])
    bho = bho.reshape(1, H)

    h, m = _tc_in(x, Win, bin2, Wm[0], bm[0].reshape(1, H))

    for k in range(L):
        partials = _sc_scatter(m, srcp, dstp, wp, zero_rows)
        p0, p1 = partials[0], partials[1]
        if k < L - 1:
            h, m = _tc_mid(h, p0, p1, Wu[k], bu[k].reshape(1, H),
                           Wm[k + 1], bm[k + 1].reshape(1, H))
        else:
            z, = _tc_out(h, p0, p1, Wu[k], bu[k].reshape(1, H), who, bho)

    return (z[:, 0], z[:, 1])


# speedup vs baseline: 6.5425x; 1.6279x over previous
import functools

import jax
import jax.numpy as jnp
from jax import lax
from jax.experimental import pallas as pl
from jax.experimental.pallas import tpu as pltpu
from jax.experimental.pallas import tpu_sc as plsc

N = 10000
E = 320000
D = 128
H = 128
L = 3

NUM_CORES = 2
NUM_SUBCORES = 16
COLS = H // NUM_CORES
GROUP = 64
G_STAGE = 32
G_SUB = 320
N_STAGES = G_SUB // G_STAGE
NBUF = 3
TRI = (G_STAGE - 2) // NBUF
EP = NUM_SUBCORES * G_SUB * GROUP
ROWS_PER_SUB = 624
ROW_TAIL = N - NUM_SUBCORES * ROWS_PER_SUB
ROW_TAIL_OFF = NUM_SUBCORES * ROWS_PER_SUB

_TC_BLOCK = 1000


def _sc_body(m_hbm, src_hbm, dst_hbm, w_hbm, zero_hbm,
             out_hbm, m_sp, acc, src_v, dst_v, att_v, rows_v,
             gs0, gs1, gs2, ss0, ss1, ss2):
    gsems = (gs0, gs1, gs2)
    ssems = (ss0, ss1, ss2)
    c = lax.axis_index("c")
    s = lax.axis_index("s")

    pltpu.sync_copy(m_hbm.at[c, pl.ds(s * ROWS_PER_SUB, ROWS_PER_SUB)],
                    m_sp.at[pl.ds(s * ROWS_PER_SUB, ROWS_PER_SUB)])
    pltpu.sync_copy(zero_hbm.at[pl.ds(0, ROWS_PER_SUB)],
                    acc.at[pl.ds(s * ROWS_PER_SUB, ROWS_PER_SUB)])

    @pl.when(s == NUM_SUBCORES - 1)
    def _stage_tail():
        pltpu.sync_copy(m_hbm.at[c, pl.ds(ROW_TAIL_OFF, ROW_TAIL)],
                        m_sp.at[pl.ds(ROW_TAIL_OFF, ROW_TAIL)])
        pltpu.sync_copy(zero_hbm.at[pl.ds(0, ROW_TAIL)],
                        acc.at[pl.ds(ROW_TAIL_OFF, ROW_TAIL)])

    def start_gather(g, b):
        pltpu.async_copy(m_sp.at[src_v.at[g]], rows_v.at[b], gsems[b])

    def wait_gather(b):
        pltpu.make_async_copy(m_sp.at[src_v.at[0]], rows_v.at[b],
                              gsems[b]).wait()

    def start_scatter(g, b):
        pltpu.async_copy(rows_v.at[b], acc.at[dst_v.at[g]], ssems[b],
                         add=True)

    def wait_scatter(b):
        pltpu.make_async_copy(rows_v.at[b], acc.at[dst_v.at[0]],
                              ssems[b]).wait()

    def scale(g, b):
        def edge_chunk_body(cidx, inner):
            e0 = cidx * 16
            av = att_v[g, pl.ds(e0, 16)]
            for j in range(16):
                a = av[j]
                for f in range(COLS // 16):
                    sl = pl.ds(f * 16, 16)
                    rows_v[b, e0 + j, sl] = rows_v[b, e0 + j, sl] * a
            return inner

        lax.fori_loop(0, GROUP // 16, edge_chunk_body, 0)

    def run_stage(base):
        pltpu.sync_copy(src_hbm.at[pl.ds(base, G_STAGE)], src_v)
        pltpu.sync_copy(dst_hbm.at[pl.ds(base, G_STAGE)], dst_v)
        pltpu.sync_copy(w_hbm.at[pl.ds(base, G_STAGE)], att_v)

        n_sl = GROUP // 16

        def sig_body(j, carry):
            g = j // n_sl
            f = (j % n_sl) * 16
            wv = att_v[g, pl.ds(f, 16)]
            att_v[g, pl.ds(f, 16)] = 1.0 / (1.0 + jnp.exp(-wv))
            return carry

        lax.fori_loop(0, G_STAGE * n_sl, sig_body, 0)

        start_gather(0, 0)
        start_gather(1, 1)

        def tri_body(i, carry):
            for b in range(NBUF):
                g = 3 * i + b
                nb = (b + 2) % 3
                if b == 0:
                    @pl.when(i > 0)
                    def _drain0():
                        wait_scatter(nb)
                else:
                    wait_scatter(nb)
                start_gather(g + 2, nb)
                wait_gather(b)
                scale(g, b)
                start_scatter(g, b)
            return carry

        lax.fori_loop(0, TRI, tri_body, 0)
        wait_scatter(2)
        wait_gather(0)
        scale(G_STAGE - 2, 0)
        start_scatter(G_STAGE - 2, 0)
        wait_gather(1)
        scale(G_STAGE - 1, 1)
        start_scatter(G_STAGE - 1, 1)
        wait_scatter(0)
        wait_scatter(1)

    plsc.subcore_barrier()

    def stage_body(hs, carry):
        run_stage(s * G_SUB + hs * G_STAGE)
        return carry

    lax.fori_loop(0, N_STAGES, stage_body, 0)

    plsc.subcore_barrier()
    pltpu.sync_copy(acc.at[pl.ds(s * ROWS_PER_SUB, ROWS_PER_SUB)],
                    out_hbm.at[c, pl.ds(s * ROWS_PER_SUB, ROWS_PER_SUB)])

    @pl.when(s == NUM_SUBCORES - 1)
    def _write_tail():
        pltpu.sync_copy(acc.at[pl.ds(ROW_TAIL_OFF, ROW_TAIL)],
                        out_hbm.at[c, pl.ds(ROW_TAIL_OFF, ROW_TAIL)])


_sc_scatter = functools.partial(
    pl.kernel,
    out_type=jax.ShapeDtypeStruct((NUM_CORES, N, COLS), jnp.float32),
    mesh=plsc.VectorSubcoreMesh(core_axis_name="c", subcore_axis_name="s"),
    scratch_types=[
        pltpu.VMEM_SHARED((N, COLS), jnp.float32),
        pltpu.VMEM_SHARED((N, COLS), jnp.float32),
        pltpu.VMEM((G_STAGE, GROUP), jnp.int32),
        pltpu.VMEM((G_STAGE, GROUP), jnp.int32),
        pltpu.VMEM((G_STAGE, GROUP), jnp.float32),
        pltpu.VMEM((NBUF, GROUP, COLS), jnp.float32),
        pltpu.SemaphoreType.DMA,
        pltpu.SemaphoreType.DMA,
        pltpu.SemaphoreType.DMA,
        pltpu.SemaphoreType.DMA,
        pltpu.SemaphoreType.DMA,
        pltpu.SemaphoreType.DMA,
    ],
)(_sc_body)


def _mm(a, b):
    return jax.lax.dot_general(a, b, (((1,), (0,)), ((), ())),
                               preferred_element_type=jnp.float32)


def _tc_in_body(x_ref, win_ref, bin_ref, wm_ref, bm_ref, h_ref, m_ref):
    h = jnp.maximum(_mm(x_ref[...], win_ref[...]) + bin_ref[...], 0.0)
    h_ref[...] = h
    m = _mm(h, wm_ref[...]) + bm_ref[...]
    m_ref[0] = m[:, :COLS]
    m_ref[1] = m[:, COLS:]


def _tc_mid_body(h_ref, p_ref, wu_ref, bu_ref, wm_ref, bm_ref,
                 h2_ref, m2_ref):
    agg = jnp.concatenate([p_ref[0], p_ref[1]], axis=1)
    t = h_ref[...] + agg
    h2 = jnp.maximum(_mm(t, wu_ref[...]) + bu_ref[...], 0.0)
    h2_ref[...] = h2
    m2 = _mm(h2, wm_ref[...]) + bm_ref[...]
    m2_ref[0] = m2[:, :COLS]
    m2_ref[1] = m2[:, COLS:]


def _tc_out_body(h_ref, p_ref, wu_ref, bu_ref, who_ref, bho_ref, z_ref):
    agg = jnp.concatenate([p_ref[0], p_ref[1]], axis=1)
    t = h_ref[...] + agg
    h3 = jnp.maximum(_mm(t, wu_ref[...]) + bu_ref[...], 0.0)
    z = _mm(h3, who_ref[...]) + bho_ref[...]
    col = lax.broadcasted_iota(jnp.int32, z.shape, 1)
    z_ref[...] = jnp.where(col == 0, jax.nn.sigmoid(z), jnp.tanh(z))


_row_spec = pl.BlockSpec((_TC_BLOCK, H), lambda i: (i, 0))
_half_spec = pl.BlockSpec((NUM_CORES, _TC_BLOCK, COLS), lambda i: (0, i, 0))
_FULL_W = pl.BlockSpec((H, H), lambda i: (0, 0))
_FULL_B = pl.BlockSpec((1, H), lambda i: (0, 0))
_GRID = (N // _TC_BLOCK,)

_m2_shape = jax.ShapeDtypeStruct((NUM_CORES, N, COLS), jnp.float32)

_tc_in = pl.pallas_call(
    _tc_in_body,
    grid=_GRID,
    in_specs=[_row_spec, _FULL_W, _FULL_B, _FULL_W, _FULL_B],
    out_specs=[_row_spec, _half_spec],
    out_shape=[jax.ShapeDtypeStruct((N, H), jnp.float32), _m2_shape],
)

_tc_mid = pl.pallas_call(
    _tc_mid_body,
    grid=_GRID,
    in_specs=[_row_spec, _half_spec, _FULL_W, _FULL_B, _FULL_W, _FULL_B],
    out_specs=[_row_spec, _half_spec],
    out_shape=[jax.ShapeDtypeStruct((N, H), jnp.float32), _m2_shape],
)

_tc_out = pl.pallas_call(
    _tc_out_body,
    grid=_GRID,
    in_specs=[_row_spec, _half_spec, _FULL_W, _FULL_B, _FULL_W, _FULL_B],
    out_specs=[_row_spec],
    out_shape=[jax.ShapeDtypeStruct((N, H), jnp.float32)],
)


def kernel(x, src, dst, w, Win, b_in, Wm, bm, Wu, bu, Wos, bos, Wside, bside):
    pad = EP - E
    srcp = jnp.concatenate([src.astype(jnp.int32),
                            jnp.zeros((pad,), jnp.int32)]).reshape(-1, GROUP)
    dstp = jnp.concatenate([dst.astype(jnp.int32),
                            jnp.zeros((pad,), jnp.int32)]).reshape(-1, GROUP)
    wp = jnp.concatenate([w, jnp.full((pad,), -1e30, jnp.float32)]
                         ).reshape(-1, GROUP)
    zero_rows = jnp.zeros((ROWS_PER_SUB, COLS), jnp.float32)

    bin2 = b_in.reshape(1, H)
    who = jnp.zeros((H, H), jnp.float32)
    who = who.at[:, 0].set(Wos[:, 0]).at[:, 1].set(Wside[:, 0])
    bho = jnp.zeros((H,), jnp.float32).at[0].set(bos[0]).at[1].set(bside[0])
    bho = bho.reshape(1, H)

    h, m = _tc_in(x, Win, bin2, Wm[0], bm[0].reshape(1, H))

    for k in range(L):
        p = _sc_scatter(m, srcp, dstp, wp, zero_rows)
        if k < L - 1:
            h, m = _tc_mid(h, p, Wu[k], bu[k].reshape(1, H),
                           Wm[k + 1], bm[k + 1].reshape(1, H))
        else:
            z, = _tc_out(h, p, Wu[k], bu[k].reshape(1, H), who, bho)

    return (z[:, 0], z[:, 1])
